# Initial kernel scaffold; baseline (speedup 1.0000x reference)
import jax
import jax.numpy as jnp
from jax.experimental import pallas as pl


def kernel(x, edge_index, W1, b1, W2, b2, fW1, fb1, fW2, fb2, fW3, fb3):
    raise NotImplementedError("write your pallas kernel here")


# speedup vs baseline: 13.9872x; 13.9872x over previous
import functools

import jax
import jax.numpy as jnp
from jax import lax
from jax.experimental import pallas as pl
from jax.experimental.pallas import tpu as pltpu
from jax.experimental.pallas import tpu_sc as plsc

NN = 10000
EE = 320000
DD = 128
HH = 64

NC = 2
NS = 16
NW = NC * NS
N_PAD = 10240
SL = N_PAD // NS
CW = 128
EW = 10240
CH = EW // CW
E_PAD = NW * EW

_SC_MESH = plsc.VectorSubcoreMesh(core_axis_name="c", subcore_axis_name="s")
_SC_PARAMS = pltpu.CompilerParams(use_tc_tiling_on_sc=False)


@functools.partial(
    pl.kernel,
    out_type=jax.ShapeDtypeStruct((NC, N_PAD, 16), jnp.float32),
    mesh=_SC_MESH,
    compiler_params=_SC_PARAMS,
    scratch_types=[
        pltpu.VMEM((CH, CW), jnp.int32),
        pltpu.VMEM((CW, 16), jnp.float32),
        pltpu.VMEM_SHARED((N_PAD, 16), jnp.float32),
    ],
)
def _deg_kernel(dst_hbm, ones_hbm, zeros_hbm, out_hbm, dstv, onesv, acc):
    c = lax.axis_index("c")
    s = lax.axis_index("s")
    wid = c * NS + s
    pltpu.sync_copy(dst_hbm.at[wid], dstv)
    pltpu.sync_copy(ones_hbm, onesv)
    pltpu.sync_copy(zeros_hbm, acc.at[pl.ds(s * SL, SL)])
    plsc.subcore_barrier()

    def body(j, carry):
        pltpu.sync_copy(onesv, acc.at[dstv.at[j]], add=True)
        return carry

    lax.fori_loop(0, CH, body, 0)
    plsc.subcore_barrier()
    pltpu.sync_copy(acc.at[pl.ds(s * SL, SL)], out_hbm.at[c, pl.ds(s * SL, SL)])


@functools.partial(
    pl.kernel,
    out_type=jax.ShapeDtypeStruct((NC, N_PAD, HH), jnp.float32),
    mesh=_SC_MESH,
    compiler_params=_SC_PARAMS,
    scratch_types=[
        pltpu.VMEM((CH, CW), jnp.int32),
        pltpu.VMEM((CH, CW), jnp.int32),
        pltpu.VMEM((CW, HH), jnp.float32),
        pltpu.VMEM_SHARED((N_PAD, HH), jnp.float32),
    ],
)
def _agg_kernel(y_hbm, src_hbm, dst_hbm, zeros_hbm, out_hbm, srcv, dstv, rows, acc):
    c = lax.axis_index("c")
    s = lax.axis_index("s")
    wid = c * NS + s
    pltpu.sync_copy(src_hbm.at[wid], srcv)
    pltpu.sync_copy(dst_hbm.at[wid], dstv)
    pltpu.sync_copy(zeros_hbm, acc.at[pl.ds(s * SL, SL)])
    plsc.subcore_barrier()

    def body(j, carry):
        pltpu.sync_copy(y_hbm.at[srcv.at[j]], rows)
        pltpu.sync_copy(rows, acc.at[dstv.at[j]], add=True)
        return carry

    lax.fori_loop(0, CH, body, 0)
    plsc.subcore_barrier()
    pltpu.sync_copy(acc.at[pl.ds(s * SL, SL)], out_hbm.at[c, pl.ds(s * SL, SL)])


R = 1024
GRID = N_PAD // R


def _t1_body(x_ref, w1_ref, dp_ref, y_ref, dinv_ref):
    deg = dp_ref[0] + dp_ref[1] + 1.0
    dinv = lax.rsqrt(deg)
    dinv_ref[...] = dinv
    xw = jnp.dot(x_ref[...], w1_ref[...], preferred_element_type=jnp.float32)
    y_ref[...] = xw * dinv[:, 0:1]


_t1 = pl.pallas_call(
    _t1_body,
    grid=(GRID,),
    in_specs=[
        pl.BlockSpec((R, DD), lambda i: (i, 0)),
        pl.BlockSpec((DD, HH), lambda i: (0, 0)),
        pl.BlockSpec((NC, R, 16), lambda i: (0, i, 0)),
    ],
    out_specs=[
        pl.BlockSpec((R, HH), lambda i: (i, 0)),
        pl.BlockSpec((R, 16), lambda i: (i, 0)),
    ],
    out_shape=[
        jax.ShapeDtypeStruct((N_PAD, HH), jnp.float32),
        jax.ShapeDtypeStruct((N_PAD, 16), jnp.float32),
    ],
)


def _t2_body(p_ref, y1_ref, dinv_ref, b1_ref, w2_ref, y2_ref):
    dinv = dinv_ref[:, 0:1]
    agg = (p_ref[0] + p_ref[1] + y1_ref[...]) * dinv + b1_ref[...]
    h = jnp.maximum(agg, 0.0)
    y2_ref[...] = jnp.dot(h, w2_ref[...], preferred_element_type=jnp.float32) * dinv


_t2 = pl.pallas_call(
    _t2_body,
    grid=(GRID,),
    in_specs=[
        pl.BlockSpec((NC, R, HH), lambda i: (0, i, 0)),
        pl.BlockSpec((R, HH), lambda i: (i, 0)),
        pl.BlockSpec((R, 16), lambda i: (i, 0)),
        pl.BlockSpec((1, HH), lambda i: (0, 0)),
        pl.BlockSpec((HH, HH), lambda i: (0, 0)),
    ],
    out_specs=pl.BlockSpec((R, HH), lambda i: (i, 0)),
    out_shape=jax.ShapeDtypeStruct((N_PAD, HH), jnp.float32),
)


def _t3_body(p_ref, y2_ref, dinv_ref, b2_ref, fw1_ref, fb1_ref, fw2_ref,
             fb2_ref, fw3_ref, fb3_ref, out_ref):
    dinv = dinv_ref[:, 0:1]
    h = jnp.maximum((p_ref[0] + p_ref[1] + y2_ref[...]) * dinv + b2_ref[...], 0.0)
    h = jnp.maximum(
        jnp.dot(h, fw1_ref[...], preferred_element_type=jnp.float32) + fb1_ref[...], 0.0)
    h = jnp.maximum(
        jnp.dot(h, fw2_ref[...], preferred_element_type=jnp.float32) + fb2_ref[...], 0.0)
    out_ref[...] = jnp.dot(h, fw3_ref[...], preferred_element_type=jnp.float32) + fb3_ref[...]


_t3 = pl.pallas_call(
    _t3_body,
    grid=(GRID,),
    in_specs=[
        pl.BlockSpec((NC, R, HH), lambda i: (0, i, 0)),
        pl.BlockSpec((R, HH), lambda i: (i, 0)),
        pl.BlockSpec((R, 16), lambda i: (i, 0)),
        pl.BlockSpec((1, HH), lambda i: (0, 0)),
        pl.BlockSpec((HH, HH), lambda i: (0, 0)),
        pl.BlockSpec((1, HH), lambda i: (0, 0)),
        pl.BlockSpec((HH, HH), lambda i: (0, 0)),
        pl.BlockSpec((1, HH), lambda i: (0, 0)),
        pl.BlockSpec((HH, 128), lambda i: (0, 0)),
        pl.BlockSpec((1, 128), lambda i: (0, 0)),
    ],
    out_specs=pl.BlockSpec((R, 128), lambda i: (i, 0)),
    out_shape=jax.ShapeDtypeStruct((N_PAD, 128), jnp.float32),
)


def kernel(x, edge_index, W1, b1, W2, b2, fW1, fb1, fW2, fb2, fW3, fb3):
    x_pad = jnp.pad(x, ((0, N_PAD - NN), (0, 0)))
    pad = jnp.full((E_PAD - EE,), NN, jnp.int32)
    src3 = jnp.concatenate([edge_index[0], pad]).reshape(NW, CH, CW)
    dst3 = jnp.concatenate([edge_index[1], pad]).reshape(NW, CH, CW)
    ones16 = jnp.ones((CW, 16), jnp.float32)
    z16 = jnp.zeros((SL, 16), jnp.float32)
    z64 = jnp.zeros((SL, HH), jnp.float32)

    degp = _deg_kernel(dst3, ones16, z16)
    y1, dinv16 = _t1(x_pad, W1, degp)
    p1 = _agg_kernel(y1, src3, dst3, z64)
    y2 = _t2(p1, y1, dinv16, b1.reshape(1, HH), W2)
    p2 = _agg_kernel(y2, src3, dst3, z64)
    fW3p = jnp.pad(fW3, ((0, 0), (0, 128 - fW3.shape[1])))
    fb3p = jnp.pad(fb3, (0, 128 - fb3.shape[0])).reshape(1, 128)
    outp = _t3(p2, y2, dinv16, b2.reshape(1, HH), fW1, fb1.reshape(1, HH),
               fW2, fb2.reshape(1, HH), fW3p, fb3p)
    return outp[:NN, 0]


# speedup vs baseline: 14.9837x; 1.0712x over previous
import functools

import jax
import jax.numpy as jnp
from jax import lax
from jax.experimental import pallas as pl
from jax.experimental.pallas import tpu as pltpu
from jax.experimental.pallas import tpu_sc as plsc

NN = 10000
EE = 320000
DD = 128
HH = 64

NC = 2
NS = 16
NW = NC * NS
N_PAD = 10240
SL = N_PAD // NS
CW = 128
EW = 10240
CH = EW // CW
E_PAD = NW * EW

_SC_MESH = plsc.VectorSubcoreMesh(core_axis_name="c", subcore_axis_name="s")
_SC_PARAMS = pltpu.CompilerParams(use_tc_tiling_on_sc=False)


@functools.partial(
    pl.kernel,
    out_type=jax.ShapeDtypeStruct((NC, N_PAD, 16), jnp.float32),
    mesh=_SC_MESH,
    compiler_params=_SC_PARAMS,
    scratch_types=[
        pltpu.VMEM((CH, CW), jnp.int32),
        pltpu.VMEM((CW, 16), jnp.float32),
        pltpu.VMEM_SHARED((N_PAD, 16), jnp.float32),
        pltpu.SemaphoreType.DMA,
    ],
)
def _deg_kernel(dst_hbm, ones_hbm, zeros_hbm, out_hbm, dstv, onesv, acc, ssem):
    c = lax.axis_index("c")
    s = lax.axis_index("s")
    wid = c * NS + s
    pltpu.sync_copy(dst_hbm.at[wid], dstv)
    pltpu.sync_copy(ones_hbm, onesv)
    pltpu.sync_copy(zeros_hbm, acc.at[pl.ds(s * SL, SL)])
    plsc.subcore_barrier()

    LAG = 8

    def ss(j):
        pltpu.async_copy(onesv, acc.at[dstv.at[j]], ssem, add=True)

    def sw(j):
        pltpu.make_async_copy(onesv, acc.at[dstv.at[j]], ssem).wait()

    for j in range(LAG):
        ss(j)

    def body(k, carry):
        ss(k + LAG)
        sw(k)
        return carry

    lax.fori_loop(0, CH - LAG, body, 0)
    for j in range(CH - LAG, CH):
        sw(j)
    plsc.subcore_barrier()
    pltpu.sync_copy(acc.at[pl.ds(s * SL, SL)], out_hbm.at[c, pl.ds(s * SL, SL)])


@functools.partial(
    pl.kernel,
    out_type=jax.ShapeDtypeStruct((NC, N_PAD, HH), jnp.float32),
    mesh=_SC_MESH,
    compiler_params=_SC_PARAMS,
    scratch_types=[
        pltpu.VMEM((CH, CW), jnp.int32),
        pltpu.VMEM((CH, CW), jnp.int32),
        pltpu.VMEM((2, CW, HH), jnp.float32),
        pltpu.VMEM_SHARED((N_PAD, HH), jnp.float32),
        pltpu.SemaphoreType.DMA,
        pltpu.SemaphoreType.DMA,
    ],
)
def _agg_kernel(y_hbm, src_hbm, dst_hbm, zeros_hbm, out_hbm, srcv, dstv, rows,
                acc, gsem, ssem):
    c = lax.axis_index("c")
    s = lax.axis_index("s")
    wid = c * NS + s
    pltpu.sync_copy(src_hbm.at[wid], srcv)
    pltpu.sync_copy(dst_hbm.at[wid], dstv)
    pltpu.sync_copy(zeros_hbm, acc.at[pl.ds(s * SL, SL)])
    plsc.subcore_barrier()

    def gs(j, b):
        pltpu.async_copy(y_hbm.at[srcv.at[j]], rows.at[b], gsem)

    def gw(j, b):
        pltpu.make_async_copy(y_hbm.at[srcv.at[j]], rows.at[b], gsem).wait()

    def ss(j, b):
        pltpu.async_copy(rows.at[b], acc.at[dstv.at[j]], ssem, add=True)

    def sw(j, b):
        pltpu.make_async_copy(rows.at[b], acc.at[dstv.at[j]], ssem).wait()

    gs(0, 0)

    def body(k, carry):
        j0 = 2 * k
        j1 = j0 + 1
        jn = lax.rem(j0 + 2, CH)
        gw(j0, 0)
        ss(j0, 0)
        gs(j1, 1)
        gw(j1, 1)
        sw(j0, 0)
        ss(j1, 1)
        gs(jn, 0)
        sw(j1, 1)
        return carry

    lax.fori_loop(0, CH // 2, body, 0)
    gw(0, 0)
    plsc.subcore_barrier()
    pltpu.sync_copy(acc.at[pl.ds(s * SL, SL)], out_hbm.at[c, pl.ds(s * SL, SL)])


R = 1024
GRID = N_PAD // R


def _t1_body(x_ref, w1_ref, dp_ref, y_ref, dinv_ref):
    deg = dp_ref[0] + dp_ref[1] + 1.0
    dinv = lax.rsqrt(deg)
    dinv_ref[...] = dinv
    xw = jnp.dot(x_ref[...], w1_ref[...], preferred_element_type=jnp.float32)
    y_ref[...] = xw * dinv[:, 0:1]


_t1 = pl.pallas_call(
    _t1_body,
    grid=(GRID,),
    in_specs=[
        pl.BlockSpec((R, DD), lambda i: (i, 0)),
        pl.BlockSpec((DD, HH), lambda i: (0, 0)),
        pl.BlockSpec((NC, R, 16), lambda i: (0, i, 0)),
    ],
    out_specs=[
        pl.BlockSpec((R, HH), lambda i: (i, 0)),
        pl.BlockSpec((R, 16), lambda i: (i, 0)),
    ],
    out_shape=[
        jax.ShapeDtypeStruct((N_PAD, HH), jnp.float32),
        jax.ShapeDtypeStruct((N_PAD, 16), jnp.float32),
    ],
)


def _t2_body(p_ref, y1_ref, dinv_ref, b1_ref, w2_ref, y2_ref):
    dinv = dinv_ref[:, 0:1]
    agg = (p_ref[0] + p_ref[1] + y1_ref[...]) * dinv + b1_ref[...]
    h = jnp.maximum(agg, 0.0)
    y2_ref[...] = jnp.dot(h, w2_ref[...], preferred_element_type=jnp.float32) * dinv


_t2 = pl.pallas_call(
    _t2_body,
    grid=(GRID,),
    in_specs=[
        pl.BlockSpec((NC, R, HH), lambda i: (0, i, 0)),
        pl.BlockSpec((R, HH), lambda i: (i, 0)),
        pl.BlockSpec((R, 16), lambda i: (i, 0)),
        pl.BlockSpec((1, HH), lambda i: (0, 0)),
        pl.BlockSpec((HH, HH), lambda i: (0, 0)),
    ],
    out_specs=pl.BlockSpec((R, HH), lambda i: (i, 0)),
    out_shape=jax.ShapeDtypeStruct((N_PAD, HH), jnp.float32),
)


def _t3_body(p_ref, y2_ref, dinv_ref, b2_ref, fw1_ref, fb1_ref, fw2_ref,
             fb2_ref, fw3_ref, fb3_ref, out_ref):
    dinv = dinv_ref[:, 0:1]
    h = jnp.maximum((p_ref[0] + p_ref[1] + y2_ref[...]) * dinv + b2_ref[...], 0.0)
    h = jnp.maximum(
        jnp.dot(h, fw1_ref[...], preferred_element_type=jnp.float32) + fb1_ref[...], 0.0)
    h = jnp.maximum(
        jnp.dot(h, fw2_ref[...], preferred_element_type=jnp.float32) + fb2_ref[...], 0.0)
    out_ref[...] = jnp.dot(h, fw3_ref[...], preferred_element_type=jnp.float32) + fb3_ref[...]


_t3 = pl.pallas_call(
    _t3_body,
    grid=(GRID,),
    in_specs=[
        pl.BlockSpec((NC, R, HH), lambda i: (0, i, 0)),
        pl.BlockSpec((R, HH), lambda i: (i, 0)),
        pl.BlockSpec((R, 16), lambda i: (i, 0)),
        pl.BlockSpec((1, HH), lambda i: (0, 0)),
        pl.BlockSpec((HH, HH), lambda i: (0, 0)),
        pl.BlockSpec((1, HH), lambda i: (0, 0)),
        pl.BlockSpec((HH, HH), lambda i: (0, 0)),
        pl.BlockSpec((1, HH), lambda i: (0, 0)),
        pl.BlockSpec((HH, 128), lambda i: (0, 0)),
        pl.BlockSpec((1, 128), lambda i: (0, 0)),
    ],
    out_specs=pl.BlockSpec((R, 128), lambda i: (i, 0)),
    out_shape=jax.ShapeDtypeStruct((N_PAD, 128), jnp.float32),
)


def kernel(x, edge_index, W1, b1, W2, b2, fW1, fb1, fW2, fb2, fW3, fb3):
    x_pad = jnp.pad(x, ((0, N_PAD - NN), (0, 0)))
    pad = jnp.full((E_PAD - EE,), NN, jnp.int32)
    src3 = jnp.concatenate([edge_index[0], pad]).reshape(NW, CH, CW)
    dst3 = jnp.concatenate([edge_index[1], pad]).reshape(NW, CH, CW)
    ones16 = jnp.ones((CW, 16), jnp.float32)
    z16 = jnp.zeros((SL, 16), jnp.float32)
    z64 = jnp.zeros((SL, HH), jnp.float32)

    degp = _deg_kernel(dst3, ones16, z16)
    y1, dinv16 = _t1(x_pad, W1, degp)
    p1 = _agg_kernel(y1, src3, dst3, z64)
    y2 = _t2(p1, y1, dinv16, b1.reshape(1, HH), W2)
    p2 = _agg_kernel(y2, src3, dst3, z64)
    fW3p = jnp.pad(fW3, ((0, 0), (0, 128 - fW3.shape[1])))
    fb3p = jnp.pad(fb3, (0, 128 - fb3.shape[0])).reshape(1, 128)
    outp = _t3(p2, y2, dinv16, b2.reshape(1, HH), fW1, fb1.reshape(1, HH),
               fW2, fb2.reshape(1, HH), fW3p, fb3p)
    return outp[:NN, 0]


# speedup vs baseline: 31.5834x; 2.1079x over previous
import functools

import jax
import jax.numpy as jnp
from jax import lax
from jax.experimental import pallas as pl
from jax.experimental.pallas import tpu as pltpu
from jax.experimental.pallas import tpu_sc as plsc

NN = 10000
EE = 320000
DD = 128
HH = 64

NC = 2
NS = 16
NW = NC * NS
N_PAD = 10240
SL = N_PAD // NS
CW = 128
EW = 10240
CH = EW // CW
E_PAD = NW * EW

_SC_MESH = plsc.VectorSubcoreMesh(core_axis_name="c", subcore_axis_name="s")
_SC_PARAMS = pltpu.CompilerParams(use_tc_tiling_on_sc=False)


@functools.partial(
    pl.kernel,
    out_type=jax.ShapeDtypeStruct((NC, N_PAD, 16), jnp.float32),
    mesh=_SC_MESH,
    compiler_params=_SC_PARAMS,
    scratch_types=[
        pltpu.VMEM((CH, CW), jnp.int32),
        pltpu.VMEM((CW, 16), jnp.float32),
        pltpu.VMEM_SHARED((N_PAD, 16), jnp.float32),
        pltpu.SemaphoreType.DMA,
    ],
)
def _deg_kernel(dst_hbm, ones_hbm, zeros_hbm, out_hbm, dstv, onesv, acc, ssem):
    c = lax.axis_index("c")
    s = lax.axis_index("s")
    wid = c * NS + s
    pltpu.sync_copy(dst_hbm.at[wid], dstv)
    pltpu.sync_copy(ones_hbm, onesv)
    pltpu.sync_copy(zeros_hbm, acc.at[pl.ds(s * SL, SL)])
    plsc.subcore_barrier()

    LAG = 8

    def ss(j):
        pltpu.async_copy(onesv, acc.at[dstv.at[j]], ssem, add=True)

    def sw(j):
        pltpu.make_async_copy(onesv, acc.at[dstv.at[j]], ssem).wait()

    for j in range(LAG):
        ss(j)

    def body(k, carry):
        ss(k + LAG)
        sw(k)
        return carry

    lax.fori_loop(0, CH - LAG, body, 0)
    for j in range(CH - LAG, CH):
        sw(j)
    plsc.subcore_barrier()
    pltpu.sync_copy(acc.at[pl.ds(s * SL, SL)], out_hbm.at[c, pl.ds(s * SL, SL)])


@functools.partial(
    pl.kernel,
    out_type=jax.ShapeDtypeStruct((NC, N_PAD, HH), jnp.float32),
    mesh=_SC_MESH,
    compiler_params=_SC_PARAMS,
    scratch_types=[
        pltpu.VMEM((CH, CW), jnp.int32),
        pltpu.VMEM((CH, CW), jnp.int32),
        pltpu.VMEM((2, CW, HH), jnp.float32),
        pltpu.VMEM_SHARED((N_PAD, HH), jnp.float32),
        pltpu.SemaphoreType.DMA,
        pltpu.SemaphoreType.DMA,
    ],
)
def _agg_kernel(y_hbm, src_hbm, dst_hbm, zeros_hbm, out_hbm, srcv, dstv, rows,
                acc, gsem, ssem):
    c = lax.axis_index("c")
    s = lax.axis_index("s")
    wid = c * NS + s
    pltpu.sync_copy(src_hbm.at[wid], srcv)
    pltpu.sync_copy(dst_hbm.at[wid], dstv)
    pltpu.sync_copy(zeros_hbm, acc.at[pl.ds(s * SL, SL)])
    plsc.subcore_barrier()

    def gs(j, b):
        pltpu.async_copy(y_hbm.at[srcv.at[j]], rows.at[b], gsem)

    def gw(j, b):
        pltpu.make_async_copy(y_hbm.at[srcv.at[j]], rows.at[b], gsem).wait()

    def ss(j, b):
        pltpu.async_copy(rows.at[b], acc.at[dstv.at[j]], ssem, add=True)

    def sw(j, b):
        pltpu.make_async_copy(rows.at[b], acc.at[dstv.at[j]], ssem).wait()

    gs(0, 0)

    def body(k, carry):
        j0 = 2 * k
        j1 = j0 + 1
        jn = lax.rem(j0 + 2, CH)
        gw(j0, 0)
        ss(j0, 0)
        gs(j1, 1)
        gw(j1, 1)
        sw(j0, 0)
        ss(j1, 1)
        gs(jn, 0)
        sw(j1, 1)
        return carry

    lax.fori_loop(0, CH // 2, body, 0)
    gw(0, 0)
    plsc.subcore_barrier()
    pltpu.sync_copy(acc.at[pl.ds(s * SL, SL)], out_hbm.at[c, pl.ds(s * SL, SL)])


R = 1024
GRID = N_PAD // R


def _t1_body(x_ref, w1_ref, dp_ref, y_ref, dinv_ref):
    deg = dp_ref[0] + dp_ref[1] + 1.0
    dinv = lax.rsqrt(deg)
    dinv_ref[...] = dinv
    xw = jnp.dot(x_ref[...], w1_ref[...], preferred_element_type=jnp.float32)
    y_ref[...] = xw * dinv[:, 0:1]


_t1 = pl.pallas_call(
    _t1_body,
    grid=(GRID,),
    in_specs=[
        pl.BlockSpec((R, DD), lambda i: (i, 0)),
        pl.BlockSpec((DD, HH), lambda i: (0, 0)),
        pl.BlockSpec((NC, R, 16), lambda i: (0, i, 0)),
    ],
    out_specs=[
        pl.BlockSpec((R, HH), lambda i: (i, 0)),
        pl.BlockSpec((R, 16), lambda i: (i, 0)),
    ],
    out_shape=[
        jax.ShapeDtypeStruct((N_PAD, HH), jnp.float32),
        jax.ShapeDtypeStruct((N_PAD, 16), jnp.float32),
    ],
)


def _t2_body(p_ref, y1_ref, dinv_ref, b1_ref, w2_ref, y2_ref):
    dinv = dinv_ref[:, 0:1]
    agg = (p_ref[0] + p_ref[1] + y1_ref[...]) * dinv + b1_ref[...]
    h = jnp.maximum(agg, 0.0)
    y2_ref[...] = jnp.dot(h, w2_ref[...], preferred_element_type=jnp.float32) * dinv


_t2 = pl.pallas_call(
    _t2_body,
    grid=(GRID,),
    in_specs=[
        pl.BlockSpec((NC, R, HH), lambda i: (0, i, 0)),
        pl.BlockSpec((R, HH), lambda i: (i, 0)),
        pl.BlockSpec((R, 16), lambda i: (i, 0)),
        pl.BlockSpec((1, HH), lambda i: (0, 0)),
        pl.BlockSpec((HH, HH), lambda i: (0, 0)),
    ],
    out_specs=pl.BlockSpec((R, HH), lambda i: (i, 0)),
    out_shape=jax.ShapeDtypeStruct((N_PAD, HH), jnp.float32),
)


def _t3_body(p_ref, y2_ref, dinv_ref, b2_ref, fw1_ref, fb1_ref, fw2_ref,
             fb2_ref, fw3_ref, fb3_ref, out_ref):
    dinv = dinv_ref[:, 0:1]
    h = jnp.maximum((p_ref[0] + p_ref[1] + y2_ref[...]) * dinv + b2_ref[...], 0.0)
    h = jnp.maximum(
        jnp.dot(h, fw1_ref[...], preferred_element_type=jnp.float32) + fb1_ref[...], 0.0)
    h = jnp.maximum(
        jnp.dot(h, fw2_ref[...], preferred_element_type=jnp.float32) + fb2_ref[...], 0.0)
    out_ref[...] = jnp.dot(h, fw3_ref[...], preferred_element_type=jnp.float32) + fb3_ref[...]


_t3 = pl.pallas_call(
    _t3_body,
    grid=(GRID,),
    in_specs=[
        pl.BlockSpec((NC, R, HH), lambda i: (0, i, 0)),
        pl.BlockSpec((R, HH), lambda i: (i, 0)),
        pl.BlockSpec((R, 16), lambda i: (i, 0)),
        pl.BlockSpec((1, HH), lambda i: (0, 0)),
        pl.BlockSpec((HH, HH), lambda i: (0, 0)),
        pl.BlockSpec((1, HH), lambda i: (0, 0)),
        pl.BlockSpec((HH, HH), lambda i: (0, 0)),
        pl.BlockSpec((1, HH), lambda i: (0, 0)),
        pl.BlockSpec((HH, 128), lambda i: (0, 0)),
        pl.BlockSpec((1, 128), lambda i: (0, 0)),
    ],
    out_specs=pl.BlockSpec((R, 128), lambda i: (i, 0)),
    out_shape=jax.ShapeDtypeStruct((N_PAD, 128), jnp.float32),
)


def kernel(x, edge_index, W1, b1, W2, b2, fW1, fb1, fW2, fb2, fW3, fb3):
    x_pad = jnp.pad(x, ((0, N_PAD - NN), (0, 0)))
    pad = NN + (jnp.arange(E_PAD - EE, dtype=jnp.int32) % (N_PAD - NN))
    src3 = jnp.concatenate([edge_index[0], pad]).reshape(NW, CH, CW)
    dst3 = jnp.concatenate([edge_index[1], pad]).reshape(NW, CH, CW)
    ones16 = jnp.ones((CW, 16), jnp.float32)
    z16 = jnp.zeros((SL, 16), jnp.float32)
    z64 = jnp.zeros((SL, HH), jnp.float32)

    degp = _deg_kernel(dst3, ones16, z16)
    y1, dinv16 = _t1(x_pad, W1, degp)
    p1 = _agg_kernel(y1, src3, dst3, z64)
    y2 = _t2(p1, y1, dinv16, b1.reshape(1, HH), W2)
    p2 = _agg_kernel(y2, src3, dst3, z64)
    fW3p = jnp.pad(fW3, ((0, 0), (0, 128 - fW3.shape[1])))
    fb3p = jnp.pad(fb3, (0, 128 - fb3.shape[0])).reshape(1, 128)
    outp = _t3(p2, y2, dinv16, b2.reshape(1, HH), fW1, fb1.reshape(1, HH),
               fW2, fb2.reshape(1, HH), fW3p, fb3p)
    return outp[:NN, 0]


# speedup vs baseline: 39.8093x; 1.2605x over previous
import functools

import jax
import jax.numpy as jnp
from jax import lax
from jax.experimental import pallas as pl
from jax.experimental.pallas import tpu as pltpu
from jax.experimental.pallas import tpu_sc as plsc

NN = 10000
EE = 320000
DD = 128
HH = 64

NC = 2
NS = 16
NW = NC * NS
N_PAD = 10240
SL = N_PAD // NS
CW = 128
EW = 10240
CH = EW // CW
E_PAD = NW * EW

_SC_MESH = plsc.VectorSubcoreMesh(core_axis_name="c", subcore_axis_name="s")
_SC_PARAMS = pltpu.CompilerParams(use_tc_tiling_on_sc=False)


@functools.partial(
    pl.kernel,
    out_type=jax.ShapeDtypeStruct((NC, N_PAD, 16), jnp.float32),
    mesh=_SC_MESH,
    compiler_params=_SC_PARAMS,
    scratch_types=[
        pltpu.VMEM((CH, CW), jnp.int32),
        pltpu.VMEM((CW, 16), jnp.float32),
        pltpu.VMEM_SHARED((N_PAD, 16), jnp.float32),
        pltpu.SemaphoreType.DMA,
    ],
)
def _deg_kernel(dst_hbm, ones_hbm, zeros_hbm, out_hbm, dstv, onesv, acc, ssem):
    c = lax.axis_index("c")
    s = lax.axis_index("s")
    wid = c * NS + s
    pltpu.sync_copy(dst_hbm.at[wid], dstv)
    pltpu.sync_copy(ones_hbm, onesv)
    pltpu.sync_copy(zeros_hbm, acc.at[pl.ds(s * SL, SL)])
    plsc.subcore_barrier()

    LAG = 8

    def ss(j):
        pltpu.async_copy(onesv, acc.at[dstv.at[j]], ssem, add=True)

    def sw(j):
        pltpu.make_async_copy(onesv, acc.at[dstv.at[j]], ssem).wait()

    for j in range(LAG):
        ss(j)

    def body(k, carry):
        ss(k + LAG)
        sw(k)
        return carry

    lax.fori_loop(0, CH - LAG, body, 0)
    for j in range(CH - LAG, CH):
        sw(j)
    plsc.subcore_barrier()
    pltpu.sync_copy(acc.at[pl.ds(s * SL, SL)], out_hbm.at[c, pl.ds(s * SL, SL)])


@functools.partial(
    pl.kernel,
    out_type=jax.ShapeDtypeStruct((NC, N_PAD, HH), jnp.float32),
    mesh=_SC_MESH,
    compiler_params=_SC_PARAMS,
    scratch_types=[
        pltpu.VMEM((CH, CW), jnp.int32),
        pltpu.VMEM((CH, CW), jnp.int32),
        pltpu.VMEM((4, CW, HH), jnp.float32),
        pltpu.VMEM_SHARED((N_PAD, HH), jnp.float32),
        pltpu.SemaphoreType.DMA,
        pltpu.SemaphoreType.DMA,
    ],
)
def _agg_kernel(y_hbm, src_hbm, dst_hbm, zeros_hbm, out_hbm, srcv, dstv, rows,
                acc, gsem, ssem):
    c = lax.axis_index("c")
    s = lax.axis_index("s")
    wid = c * NS + s
    pltpu.sync_copy(src_hbm.at[wid], srcv)
    pltpu.sync_copy(dst_hbm.at[wid], dstv)
    pltpu.sync_copy(zeros_hbm, acc.at[pl.ds(s * SL, SL)])
    plsc.subcore_barrier()

    def gs(j):
        pltpu.async_copy(y_hbm.at[srcv.at[j]], rows.at[lax.rem(j, 4)], gsem)

    def gw(j):
        pltpu.make_async_copy(
            y_hbm.at[srcv.at[j]], rows.at[lax.rem(j, 4)], gsem).wait()

    def ss(j):
        pltpu.async_copy(rows.at[lax.rem(j, 4)], acc.at[dstv.at[j]], ssem,
                         add=True)

    def sw(j):
        pltpu.make_async_copy(
            rows.at[lax.rem(j, 4)], acc.at[dstv.at[j]], ssem).wait()

    gs(0)
    gs(1)
    gw(0)
    ss(0)
    gs(2)
    gw(1)
    ss(1)
    gs(3)

    def body(k, carry):
        j = k + 2
        gw(j)
        ss(j)
        sw(j - 2)
        gs(j + 2)
        return carry

    lax.fori_loop(0, CH - 4, body, 0)
    for j in range(CH - 2, CH):
        gw(j)
        ss(j)
        sw(j - 2)
    sw(CH - 2)
    sw(CH - 1)
    plsc.subcore_barrier()
    pltpu.sync_copy(acc.at[pl.ds(s * SL, SL)], out_hbm.at[c, pl.ds(s * SL, SL)])


R = 1024
GRID = N_PAD // R


def _t1_body(x_ref, w1_ref, dp_ref, y_ref, dinv_ref):
    deg = dp_ref[0] + dp_ref[1] + 1.0
    dinv = lax.rsqrt(deg)
    dinv_ref[...] = dinv
    xw = jnp.dot(x_ref[...], w1_ref[...], preferred_element_type=jnp.float32)
    y_ref[...] = xw * dinv[:, 0:1]


_t1 = pl.pallas_call(
    _t1_body,
    grid=(GRID,),
    in_specs=[
        pl.BlockSpec((R, DD), lambda i: (i, 0)),
        pl.BlockSpec((DD, HH), lambda i: (0, 0)),
        pl.BlockSpec((NC, R, 16), lambda i: (0, i, 0)),
    ],
    out_specs=[
        pl.BlockSpec((R, HH), lambda i: (i, 0)),
        pl.BlockSpec((R, 16), lambda i: (i, 0)),
    ],
    out_shape=[
        jax.ShapeDtypeStruct((N_PAD, HH), jnp.float32),
        jax.ShapeDtypeStruct((N_PAD, 16), jnp.float32),
    ],
)


def _t2_body(p_ref, y1_ref, dinv_ref, b1_ref, w2_ref, y2_ref):
    dinv = dinv_ref[:, 0:1]
    agg = (p_ref[0] + p_ref[1] + y1_ref[...]) * dinv + b1_ref[...]
    h = jnp.maximum(agg, 0.0)
    y2_ref[...] = jnp.dot(h, w2_ref[...], preferred_element_type=jnp.float32) * dinv


_t2 = pl.pallas_call(
    _t2_body,
    grid=(GRID,),
    in_specs=[
        pl.BlockSpec((NC, R, HH), lambda i: (0, i, 0)),
        pl.BlockSpec((R, HH), lambda i: (i, 0)),
        pl.BlockSpec((R, 16), lambda i: (i, 0)),
        pl.BlockSpec((1, HH), lambda i: (0, 0)),
        pl.BlockSpec((HH, HH), lambda i: (0, 0)),
    ],
    out_specs=pl.BlockSpec((R, HH), lambda i: (i, 0)),
    out_shape=jax.ShapeDtypeStruct((N_PAD, HH), jnp.float32),
)


def _t3_body(p_ref, y2_ref, dinv_ref, b2_ref, fw1_ref, fb1_ref, fw2_ref,
             fb2_ref, fw3_ref, fb3_ref, out_ref):
    dinv = dinv_ref[:, 0:1]
    h = jnp.maximum((p_ref[0] + p_ref[1] + y2_ref[...]) * dinv + b2_ref[...], 0.0)
    h = jnp.maximum(
        jnp.dot(h, fw1_ref[...], preferred_element_type=jnp.float32) + fb1_ref[...], 0.0)
    h = jnp.maximum(
        jnp.dot(h, fw2_ref[...], preferred_element_type=jnp.float32) + fb2_ref[...], 0.0)
    out_ref[...] = jnp.dot(h, fw3_ref[...], preferred_element_type=jnp.float32) + fb3_ref[...]


_t3 = pl.pallas_call(
    _t3_body,
    grid=(GRID,),
    in_specs=[
        pl.BlockSpec((NC, R, HH), lambda i: (0, i, 0)),
        pl.BlockSpec((R, HH), lambda i: (i, 0)),
        pl.BlockSpec((R, 16), lambda i: (i, 0)),
        pl.BlockSpec((1, HH), lambda i: (0, 0)),
        pl.BlockSpec((HH, HH), lambda i: (0, 0)),
        pl.BlockSpec((1, HH), lambda i: (0, 0)),
        pl.BlockSpec((HH, HH), lambda i: (0, 0)),
        pl.BlockSpec((1, HH), lambda i: (0, 0)),
        pl.BlockSpec((HH, 128), lambda i: (0, 0)),
        pl.BlockSpec((1, 128), lambda i: (0, 0)),
    ],
    out_specs=pl.BlockSpec((R, 128), lambda i: (i, 0)),
    out_shape=jax.ShapeDtypeStruct((N_PAD, 128), jnp.float32),
)


def kernel(x, edge_index, W1, b1, W2, b2, fW1, fb1, fW2, fb2, fW3, fb3):
    x_pad = jnp.pad(x, ((0, N_PAD - NN), (0, 0)))
    pad = NN + (jnp.arange(E_PAD - EE, dtype=jnp.int32) % (N_PAD - NN))
    src3 = jnp.concatenate([edge_index[0], pad]).reshape(NW, CH, CW)
    dst3 = jnp.concatenate([edge_index[1], pad]).reshape(NW, CH, CW)
    ones16 = jnp.ones((CW, 16), jnp.float32)
    z16 = jnp.zeros((SL, 16), jnp.float32)
    z64 = jnp.zeros((SL, HH), jnp.float32)

    degp = _deg_kernel(dst3, ones16, z16)
    y1, dinv16 = _t1(x_pad, W1, degp)
    p1 = _agg_kernel(y1, src3, dst3, z64)
    y2 = _t2(p1, y1, dinv16, b1.reshape(1, HH), W2)
    p2 = _agg_kernel(y2, src3, dst3, z64)
    fW3p = jnp.pad(fW3, ((0, 0), (0, 128 - fW3.shape[1])))
    fb3p = jnp.pad(fb3, (0, 128 - fb3.shape[0])).reshape(1, 128)
    outp = _t3(p2, y2, dinv16, b2.reshape(1, HH), fW1, fb1.reshape(1, HH),
               fW2, fb2.reshape(1, HH), fW3p, fb3p)
    return outp[:NN, 0]
